# Initial kernel scaffold; baseline (speedup 1.0000x reference)
import jax
import jax.numpy as jnp
from jax.experimental import pallas as pl


def kernel(x, seq_pc_feature, t_pos_embeding, t_feature_edge, edge_index, batch, lin_pc_w, lin_pc_b, lin_x_w, lin_x_b, lin_x1_w, lin_x1_b, lin_x2_w, lin_x2_b, lin_pos1_w, lin_pos1_b, lin_pos2_w, lin_pos2_b, lin_edge1_w, lin_edge1_b, lin_edge2_w, lin_edge2_b, c0_lin_l_w, c0_lin_l_b, c0_lin_r_w, c0_lin_r_b, c1_lin_l_w, c1_lin_l_b, c1_lin_r_w, c1_lin_r_b, cat0_w, cat0_b, cat1_w, cat1_b, final_w, final_b):
    raise NotImplementedError("write your pallas kernel here")


# speedup vs baseline: 2.1608x; 2.1608x over previous
import functools

import jax
import jax.numpy as jnp
from jax import lax
from jax.experimental import pallas as pl
from jax.experimental.pallas import tpu as pltpu
from jax.experimental.pallas import tpu_sc as plsc

N = 10000
E = 320000
H = 128


_C = 80
_ESUB = E // 16
_NCHUNK = _ESUB // _C
_NP = 10240
_NSUB = _NP // 16


def _sc_body(x1_hbm, fcat_hbm, src_hbm, dst_hbm, out_hbm,
             src_v, dst_v, xrows, frows, zbuf, agg_sp, gsem):
    cid = lax.axis_index("c")
    sid = lax.axis_index("s")

    def zero_row(r, carry):
        for g in range(8):
            zbuf[r, pl.ds(g * 16, 16)] = jnp.zeros((16,), jnp.float32)
        return carry

    lax.fori_loop(0, 128, zero_row, 0, unroll=False)
    for k in range(5):
        pltpu.sync_copy(zbuf, agg_sp.at[pl.ds(sid * _NSUB + k * 128, 128)])
    plsc.subcore_barrier()

    def chunk_body(j, carry):
        off = sid * _ESUB + j * _C
        pltpu.sync_copy(src_hbm.at[pl.ds(off, _C)], src_v)
        pltpu.sync_copy(dst_hbm.at[pl.ds(off, _C)], dst_v)
        cp = pltpu.async_copy(x1_hbm.at[src_v], xrows, gsem)
        pltpu.sync_copy(fcat_hbm.at[pl.ds(cid * E + off, _C)], frows)
        cp.wait()

        def mul_row(r, c2):
            for g in range(8):
                sl = pl.ds(g * 16, 16)
                frows[r, sl] = frows[r, sl] * xrows[r, sl]
            return c2

        lax.fori_loop(0, _C, mul_row, 0, unroll=False)
        pltpu.sync_copy(frows, agg_sp.at[dst_v], add=True)
        return carry

    lax.fori_loop(0, _NCHUNK, chunk_body, 0, unroll=False)
    plsc.subcore_barrier()

    pltpu.sync_copy(agg_sp.at[pl.ds(sid * _NSUB, _NSUB)],
                    out_hbm.at[pl.ds(cid * _NP + sid * _NSUB, _NSUB)])


@jax.jit
def _sc_dual_segsum(x1, fcat, src, dst):
    kern = pl.kernel(
        _sc_body,
        out_type=jax.ShapeDtypeStruct((2 * _NP, H), jnp.float32),
        mesh=plsc.VectorSubcoreMesh(core_axis_name="c", subcore_axis_name="s"),
        scratch_types=[
            pltpu.VMEM((_C,), jnp.int32),
            pltpu.VMEM((_C,), jnp.int32),
            pltpu.VMEM((_C, H), jnp.float32),
            pltpu.VMEM((_C, H), jnp.float32),
            pltpu.VMEM((128, H), jnp.float32),
            pltpu.VMEM_SHARED((_NP, H), jnp.float32),
            pltpu.SemaphoreType.DMA,
        ],
    )
    return kern(x1, fcat, src, dst)


_RB = 1000
_EB = 4000


def _pre_body(x_ref, w1_ref, b1_ref, w2_ref, b2_ref, tpc_ref, pe_ref):
    t = jax.nn.relu(jnp.dot(x_ref[...], w1_ref[...],
                            preferred_element_type=jnp.float32) + b1_ref[...])
    tpc_ref[...] = t
    pe_ref[...] = jax.nn.relu(jnp.dot(t, w2_ref[...],
                                      preferred_element_type=jnp.float32) + b2_ref[...])


def _tc_pre(x, w1, b1, w2, b2):
    full = pl.BlockSpec((H, H), lambda i: (0, 0))
    bias = pl.BlockSpec((1, H), lambda i: (0, 0))
    row = pl.BlockSpec((_RB, H), lambda i: (i, 0))
    return pl.pallas_call(
        _pre_body,
        grid=(N // _RB,),
        in_specs=[row, full, bias, full, bias],
        out_specs=[row, row],
        out_shape=[jax.ShapeDtypeStruct((N, H), jnp.float32)] * 2,
    )(x, w1, b1, w2, b2)


def _xf_body(pe_ref, w1_ref, b1_ref, w2_ref, b2_ref, x1_ref, x2_ref):
    p = pe_ref[...]
    x1_ref[...] = jax.nn.relu(jnp.dot(p, w1_ref[...],
                                      preferred_element_type=jnp.float32) + b1_ref[...])
    x2_ref[...] = jax.nn.relu(jnp.dot(p, w2_ref[...],
                                      preferred_element_type=jnp.float32) + b2_ref[...])


def _tc_xf(pe, w1, b1, w2, b2):
    full = pl.BlockSpec((H, H), lambda i: (0, 0))
    bias = pl.BlockSpec((1, H), lambda i: (0, 0))
    row = pl.BlockSpec((_RB, H), lambda i: (i, 0))
    return pl.pallas_call(
        _xf_body,
        grid=(N // _RB,),
        in_specs=[row, full, bias, full, bias],
        out_specs=[row, row],
        out_shape=[jax.ShapeDtypeStruct((N, H), jnp.float32)] * 2,
    )(pe, w1, b1, w2, b2)


def _edge_body(e_ref, w_ref, b_ref, f_ref):
    f_ref[0] = jnp.dot(e_ref[0], w_ref[0],
                       preferred_element_type=jnp.float32) + b_ref[0]


def _tc_edge(ecat, wstack, bstack):
    return pl.pallas_call(
        _edge_body,
        grid=(2, E // _EB),
        in_specs=[
            pl.BlockSpec((1, _EB, 16), lambda c, i: (c, i, 0)),
            pl.BlockSpec((1, 16, H), lambda c, i: (c, 0, 0)),
            pl.BlockSpec((1, 1, H), lambda c, i: (c, 0, 0)),
        ],
        out_specs=pl.BlockSpec((1, _EB, H), lambda c, i: (c, i, 0)),
        out_shape=jax.ShapeDtypeStruct((2, E, H), jnp.float32),
    )(ecat, wstack, bstack)


def _post_body(agg_ref, x1_ref, x2_ref, pe_ref,
               lw0_ref, rw0_ref, p2w_ref, lw1_ref, rw1_ref, e2w_ref,
               c0a_ref, c0b_ref, c1w_ref, fw_ref,
               lb0_ref, p2b_ref, lb1_ref, e2b_ref, c0bias_ref, c1bias_ref,
               fbias_ref, out_ref):
    dot = functools.partial(jnp.dot, preferred_element_type=jnp.float32)
    x1 = x1_ref[...]
    h0 = dot(agg_ref[0], lw0_ref[...]) + dot(x1, rw0_ref[...]) + lb0_ref[...]
    h0 = jax.nn.relu(dot(h0, p2w_ref[...]) + p2b_ref[...])
    h1 = dot(agg_ref[1], lw1_ref[...]) + dot(x1, rw1_ref[...]) + lb1_ref[...]
    h1 = jax.nn.relu(dot(h1, e2w_ref[...]) + e2b_ref[...])
    h = jax.nn.relu(dot(h0, c0a_ref[...]) + dot(h1, c0b_ref[...]) + c0bias_ref[...])
    h = jax.nn.relu(dot(h, c1w_ref[...]) + c1bias_ref[...])
    h = h + x2_ref[...]
    out_ref[...] = dot(h, fw_ref[...]) + fbias_ref[...] + pe_ref[...]


def _tc_post(agg, x1, x2, pe, mats, biases):
    full = pl.BlockSpec((H, H), lambda i: (0, 0))
    bias = pl.BlockSpec((1, H), lambda i: (0, 0))
    row = pl.BlockSpec((_RB, H), lambda i: (i, 0))
    aggs = pl.BlockSpec((2, _RB, H), lambda i: (0, i, 0))
    return pl.pallas_call(
        _post_body,
        grid=(N // _RB,),
        in_specs=[aggs, row, row, row] + [full] * len(mats) + [bias] * len(biases),
        out_specs=row,
        out_shape=jax.ShapeDtypeStruct((N, H), jnp.float32),
    )(agg, x1, x2, pe, *mats, *biases)


def kernel(x, seq_pc_feature, t_pos_embeding, t_feature_edge, edge_index, batch,
           lin_pc_w, lin_pc_b, lin_x_w, lin_x_b, lin_x1_w, lin_x1_b, lin_x2_w,
           lin_x2_b, lin_pos1_w, lin_pos1_b, lin_pos2_w, lin_pos2_b, lin_edge1_w,
           lin_edge1_b, lin_edge2_w, lin_edge2_b, c0_lin_l_w, c0_lin_l_b,
           c0_lin_r_w, c0_lin_r_b, c1_lin_l_w, c1_lin_l_b, c1_lin_r_w, c1_lin_r_b,
           cat0_w, cat0_b, cat1_w, cat1_b, final_w, final_b):
    r1 = lambda v: v.reshape(1, H)
    src = edge_index[0]
    dst = edge_index[1]

    t_pc, pe = _tc_pre(x, lin_pc_w, r1(lin_pc_b), lin_x_w, r1(lin_x_b))

    ecat = jnp.stack([t_pos_embeding, t_feature_edge])

    for b in range(2):
        wstack = jnp.stack([lin_pos1_w[b], lin_edge1_w[b]])
        bstack = jnp.stack([lin_pos1_b[b], lin_edge1_b[b]]).reshape(2, 1, H)
        fcat = _tc_edge(ecat, wstack, bstack).reshape(2 * E, H)

        x1, x2 = _tc_xf(pe, lin_x1_w[b], r1(lin_x1_b[b]),
                        lin_x2_w[b], r1(lin_x2_b[b]))

        agg = _sc_dual_segsum(x1, fcat, src, dst).reshape(2, _NP, H)

        mats = [c0_lin_l_w[b], c0_lin_r_w[b], lin_pos2_w[b],
                c1_lin_l_w[b], c1_lin_r_w[b], lin_edge2_w[b],
                cat0_w[b][:H], cat0_w[b][H:], cat1_w[b], final_w[b]]
        biases = [r1(c0_lin_l_b[b] + c0_lin_r_b[b]), r1(lin_pos2_b[b]),
                  r1(c1_lin_l_b[b] + c1_lin_r_b[b]), r1(lin_edge2_b[b]),
                  r1(cat0_b[b]), r1(cat1_b[b]), r1(final_b[b])]
        pe = _tc_post(agg, x1, x2, pe, mats, biases)

    return jnp.concatenate([t_pc, pe], axis=1)
